# baseline (speedup 1.0000x reference)
import functools

import jax
import jax.numpy as jnp
from jax import lax
from jax.experimental import pallas as pl
from jax.experimental.pallas import tpu as pltpu
from jax.experimental.pallas import tpu_sc as plsc

D = 1024
E = 8
EP = 128
F = 2048
T = 2048
K = 2
BM = 256
NB = 23
PMAX = NB * BM
SQRT1_2 = 0.7071067811865476

NC, NS = 2, 16
NW = NC * NS
TW = T // NW
CW = 32


def _router_body(x_ref, wr_ref, r0_ref, r1_ref, w0_ref, w1_ref, gid_ref, nu_ref):
    xx = x_ref[...]
    L = jnp.dot(xx, wr_ref[...], preferred_element_type=jnp.float32)
    idx = lax.broadcasted_iota(jnp.int32, (T, EP), 1)
    L = jnp.where(idx < E, L, -1e30)
    m1 = jnp.max(L, axis=1, keepdims=True)
    e0 = jnp.min(jnp.where(L >= m1, idx, 2**30), axis=1, keepdims=True)
    L2 = jnp.where(idx == e0, -1e30, L)
    m2 = jnp.max(L2, axis=1, keepdims=True)
    e1 = jnp.min(jnp.where(L2 >= m2, idx, 2**30), axis=1, keepdims=True)
    w0 = 1.0 / (1.0 + jnp.exp(m2 - m1))
    w1 = 1.0 - w0

    oh0 = (idx == e0).astype(jnp.float32)
    oh1 = (idx == e1).astype(jnp.float32)
    mask = oh0 + oh1

    ri = lax.broadcasted_iota(jnp.int32, (BM, BM), 0)
    ci = lax.broadcasted_iota(jnp.int32, (BM, BM), 1)
    tril = (ri > ci).astype(jnp.float32)
    parts = []
    carry = jnp.zeros((1, EP), dtype=jnp.float32)
    for i in range(T // BM):
        blk = mask[i * BM:(i + 1) * BM, :]
        parts.append(jnp.dot(tril, blk, preferred_element_type=jnp.float32) + carry)
        carry = carry + jnp.sum(blk, axis=0, keepdims=True)
    rank = jnp.concatenate(parts, axis=0)
    n = carry

    nb = jnp.floor((n + (BM - 1)) / BM)
    padded = nb * BM
    li = lax.broadcasted_iota(jnp.int32, (EP, EP), 0)
    lj = lax.broadcasted_iota(jnp.int32, (EP, EP), 1)
    u_lt = (li < lj).astype(jnp.float32)
    u_le = (li <= lj).astype(jnp.float32)
    off = jnp.dot(padded, u_lt, preferred_element_type=jnp.float32)
    cnb = jnp.dot(nb, u_le, preferred_element_type=jnp.float32)

    dest = off + rank
    r0 = jnp.sum(oh0 * dest, axis=1, keepdims=True)
    r1 = jnp.sum(oh1 * dest, axis=1, keepdims=True)
    r0_ref[...] = r0.astype(jnp.int32)
    r1_ref[...] = r1.astype(jnp.int32)
    w0_ref[...] = jnp.broadcast_to(w0, (T, 16))
    w1_ref[...] = jnp.broadcast_to(w1, (T, 16))

    brow = lax.broadcasted_iota(jnp.int32, (EP, EP), 0).astype(jnp.float32)
    cmp = (jnp.broadcast_to(cnb, (EP, EP)) <= brow).astype(jnp.float32)
    g = jnp.sum(cmp, axis=1, keepdims=True)
    lane = lax.broadcasted_iota(jnp.int32, (1, EP), 1)
    g_last = jnp.max(jnp.where((n > 0) & (lane < E), lane, 0), axis=1, keepdims=True)
    g = jnp.minimum(g, g_last.astype(jnp.float32))
    gid_ref[...] = g.astype(jnp.int32)
    nu_ref[...] = jnp.sum(nb, axis=1, keepdims=True).astype(jnp.int32)


def _router_call(xf, Wrp, interpret=False):
    return pl.pallas_call(
        _router_body,
        out_shape=[
            jax.ShapeDtypeStruct((T, 1), jnp.int32),
            jax.ShapeDtypeStruct((T, 1), jnp.int32),
            jax.ShapeDtypeStruct((T, 16), jnp.float32),
            jax.ShapeDtypeStruct((T, 16), jnp.float32),
            jax.ShapeDtypeStruct((EP, 1), jnp.int32),
            jax.ShapeDtypeStruct((1, 1), jnp.int32),
        ],
        interpret=interpret,
    )(xf, Wrp)


def _sc_scatter_body(x_hbm, r0_hbm, r1_hbm, xg_hbm, i0v, i1v, xv, sem):
    wid = lax.axis_index("s") * NC + lax.axis_index("c")
    base = wid * TW
    pltpu.sync_copy(x_hbm.at[pl.ds(base, TW)], xv)
    pltpu.sync_copy(r0_hbm.at[pl.ds(base, TW)], i0v)
    pltpu.sync_copy(r1_hbm.at[pl.ds(base, TW)], i1v)
    pltpu.async_copy(xv, xg_hbm.at[i0v], sem).wait()
    pltpu.async_copy(xv, xg_hbm.at[i1v], sem).wait()


def _sc_scatter(xf, r0, r1):
    mesh = plsc.VectorSubcoreMesh(core_axis_name="c", subcore_axis_name="s")
    fn = functools.partial(
        pl.kernel,
        mesh=mesh,
        out_type=jax.ShapeDtypeStruct((PMAX, D), jnp.float32),
        scratch_types=[
            pltpu.VMEM((TW,), jnp.int32),
            pltpu.VMEM((TW,), jnp.int32),
            pltpu.VMEM((TW, D), jnp.float32),
            pltpu.SemaphoreType.DMA,
        ],
    )(_sc_scatter_body)
    return fn(xf, r0, r1)


def _ffn_body(gid_ref, nu_ref, x_ref, w1_ref, b1_ref, w2_ref, b2_ref, y_ref):
    i = pl.program_id(0)

    @pl.when(i < nu_ref[0])
    def _():
        xb = x_ref[...].astype(jnp.bfloat16)
        h = jnp.dot(xb, w1_ref[0], preferred_element_type=jnp.float32) + b1_ref[0]
        h = 0.5 * h * (1.0 + lax.erf(h * SQRT1_2))
        y_ref[...] = jnp.dot(h.astype(jnp.bfloat16), w2_ref[0],
                             preferred_element_type=jnp.float32) + b2_ref[0]


def _ffn_call(gid, nu, xg, W1, b1, W2, b2, interpret=False):
    grid_spec = pltpu.PrefetchScalarGridSpec(
        num_scalar_prefetch=2,
        grid=(NB,),
        in_specs=[
            pl.BlockSpec((BM, D), lambda i, gid, nu: (i, 0)),
            pl.BlockSpec((1, D, F), lambda i, gid, nu: (gid[i], 0, 0)),
            pl.BlockSpec((1, 1, F), lambda i, gid, nu: (gid[i], 0, 0)),
            pl.BlockSpec((1, F, D), lambda i, gid, nu: (gid[i], 0, 0)),
            pl.BlockSpec((1, 1, D), lambda i, gid, nu: (gid[i], 0, 0)),
        ],
        out_specs=pl.BlockSpec((BM, D), lambda i, gid, nu: (i, 0)),
    )
    return pl.pallas_call(
        _ffn_body,
        grid_spec=grid_spec,
        out_shape=jax.ShapeDtypeStruct((PMAX, D), jnp.float32),
        interpret=interpret,
    )(gid, nu, xg, W1.astype(jnp.bfloat16), b1.reshape(E, 1, F),
      W2.astype(jnp.bfloat16), b2.reshape(E, 1, D))


def _sc_combine_body(y_hbm, r0_hbm, r1_hbm, w0_hbm, w1_hbm, out_hbm,
                     i0v, i1v, w0v, w1v, y0v, y1v, sem):
    wid = lax.axis_index("s") * NC + lax.axis_index("c")
    base = wid * TW
    for c in range(TW // CW):
        cb = base + c * CW
        pltpu.sync_copy(r0_hbm.at[pl.ds(cb, CW)], i0v)
        pltpu.sync_copy(r1_hbm.at[pl.ds(cb, CW)], i1v)
        pltpu.sync_copy(w0_hbm.at[pl.ds(cb, CW)], w0v)
        pltpu.sync_copy(w1_hbm.at[pl.ds(cb, CW)], w1v)
        pltpu.async_copy(y_hbm.at[i0v], y0v, sem).wait()
        pltpu.async_copy(y_hbm.at[i1v], y1v, sem).wait()

        def tok_body(i, _):
            a = w0v[i, :]
            b = w1v[i, :]

            def col_body(j, _):
                sl = pl.ds(j * 16, 16)
                y0v[i, sl] = y0v[i, sl] * a + y1v[i, sl] * b
                return 0

            lax.fori_loop(0, D // 16, col_body, 0)
            return 0

        lax.fori_loop(0, CW, tok_body, 0)
        pltpu.sync_copy(y0v, out_hbm.at[pl.ds(cb, CW)])


def _sc_combine(y, r0, r1, w0r, w1r):
    mesh = plsc.VectorSubcoreMesh(core_axis_name="c", subcore_axis_name="s")
    fn = functools.partial(
        pl.kernel,
        mesh=mesh,
        out_type=jax.ShapeDtypeStruct((T, D), jnp.float32),
        scratch_types=[
            pltpu.VMEM((CW,), jnp.int32),
            pltpu.VMEM((CW,), jnp.int32),
            pltpu.VMEM((CW, 16), jnp.float32),
            pltpu.VMEM((CW, 16), jnp.float32),
            pltpu.VMEM((CW, D), jnp.float32),
            pltpu.VMEM((CW, D), jnp.float32),
            pltpu.SemaphoreType.DMA,
        ],
    )(_sc_combine_body)
    return fn(y, r0, r1, w0r, w1r)


def kernel(x, Wr, W1, b1, W2, b2):
    B, S, d = x.shape
    xf = x.reshape(B * S, d)
    Wrp = jnp.pad(Wr, ((0, 0), (0, EP - E)))
    r0, r1, w0r, w1r, gid, nu = _router_call(xf, Wrp)
    r0 = r0.reshape(T)
    r1 = r1.reshape(T)
    xg = _sc_scatter(xf, r0, r1)
    y = _ffn_call(gid[:NB, 0], nu.reshape(1), xg, W1, b1, W2, b2)
    out = _sc_combine(y, r0, r1, w0r, w1r)
    return out.reshape(B, S, d), jnp.array(0.0, dtype=x.dtype)


# speedup vs baseline: 1.2105x; 1.2105x over previous
import functools

import jax
import jax.numpy as jnp
from jax import lax
from jax.experimental import pallas as pl
from jax.experimental.pallas import tpu as pltpu
from jax.experimental.pallas import tpu_sc as plsc

D = 1024
E = 8
EP = 128
F = 2048
T = 2048
K = 2
BM = 128
NB = 39
PMAX = NB * BM
SQRT1_2 = 0.7071067811865476

NC, NS = 2, 16
NW = NC * NS
TW = T // NW
CW = 32


def _router_body(x_ref, wr_ref, r0_ref, r1_ref, w0_ref, w1_ref, gid_ref, nu_ref):
    xx = x_ref[...]
    L = jnp.dot(xx, wr_ref[...], preferred_element_type=jnp.float32)
    idx = lax.broadcasted_iota(jnp.int32, (T, EP), 1)
    L = jnp.where(idx < E, L, -1e30)
    m1 = jnp.max(L, axis=1, keepdims=True)
    e0 = jnp.min(jnp.where(L >= m1, idx, 2**30), axis=1, keepdims=True)
    L2 = jnp.where(idx == e0, -1e30, L)
    m2 = jnp.max(L2, axis=1, keepdims=True)
    e1 = jnp.min(jnp.where(L2 >= m2, idx, 2**30), axis=1, keepdims=True)
    w0 = 1.0 / (1.0 + jnp.exp(m2 - m1))
    w1 = 1.0 - w0

    oh0 = (idx == e0).astype(jnp.float32)
    oh1 = (idx == e1).astype(jnp.float32)
    mask = oh0 + oh1

    ri = lax.broadcasted_iota(jnp.int32, (BM, BM), 0)
    ci = lax.broadcasted_iota(jnp.int32, (BM, BM), 1)
    tril = (ri > ci).astype(jnp.float32)
    parts = []
    carry = jnp.zeros((1, EP), dtype=jnp.float32)
    for i in range(T // BM):
        blk = mask[i * BM:(i + 1) * BM, :]
        parts.append(jnp.dot(tril, blk, preferred_element_type=jnp.float32) + carry)
        carry = carry + jnp.sum(blk, axis=0, keepdims=True)
    rank = jnp.concatenate(parts, axis=0)
    n = carry

    nb = jnp.floor((n + (BM - 1)) / BM)
    padded = nb * BM
    li = lax.broadcasted_iota(jnp.int32, (EP, EP), 0)
    lj = lax.broadcasted_iota(jnp.int32, (EP, EP), 1)
    u_lt = (li < lj).astype(jnp.float32)
    u_le = (li <= lj).astype(jnp.float32)
    off = jnp.dot(padded, u_lt, preferred_element_type=jnp.float32)
    cnb = jnp.dot(nb, u_le, preferred_element_type=jnp.float32)

    dest = off + rank
    r0 = jnp.sum(oh0 * dest, axis=1, keepdims=True)
    r1 = jnp.sum(oh1 * dest, axis=1, keepdims=True)
    r0_ref[...] = r0.astype(jnp.int32)
    r1_ref[...] = r1.astype(jnp.int32)
    w0_ref[...] = jnp.broadcast_to(w0, (T, 16))
    w1_ref[...] = jnp.broadcast_to(w1, (T, 16))

    brow = lax.broadcasted_iota(jnp.int32, (EP, EP), 0).astype(jnp.float32)
    cmp = (jnp.broadcast_to(cnb, (EP, EP)) <= brow).astype(jnp.float32)
    g = jnp.sum(cmp, axis=1, keepdims=True)
    lane = lax.broadcasted_iota(jnp.int32, (1, EP), 1)
    g_last = jnp.max(jnp.where((n > 0) & (lane < E), lane, 0), axis=1, keepdims=True)
    g = jnp.minimum(g, g_last.astype(jnp.float32))
    gid_ref[...] = g.astype(jnp.int32)
    nu_ref[...] = jnp.sum(nb, axis=1, keepdims=True).astype(jnp.int32)


def _router_call(xf, Wrp, interpret=False):
    return pl.pallas_call(
        _router_body,
        out_shape=[
            jax.ShapeDtypeStruct((T, 1), jnp.int32),
            jax.ShapeDtypeStruct((T, 1), jnp.int32),
            jax.ShapeDtypeStruct((T, 16), jnp.float32),
            jax.ShapeDtypeStruct((T, 16), jnp.float32),
            jax.ShapeDtypeStruct((EP, 1), jnp.int32),
            jax.ShapeDtypeStruct((1, 1), jnp.int32),
        ],
        interpret=interpret,
    )(xf, Wrp)


def _sc_scatter_body(x_hbm, r0_hbm, r1_hbm, xg_hbm, i0v, i1v, xv, sem):
    wid = lax.axis_index("s") * NC + lax.axis_index("c")
    base = wid * TW
    pltpu.sync_copy(x_hbm.at[pl.ds(base, TW)], xv)
    pltpu.sync_copy(r0_hbm.at[pl.ds(base, TW)], i0v)
    pltpu.sync_copy(r1_hbm.at[pl.ds(base, TW)], i1v)
    pltpu.async_copy(xv, xg_hbm.at[i0v], sem).wait()
    pltpu.async_copy(xv, xg_hbm.at[i1v], sem).wait()


def _sc_scatter(xf, r0, r1):
    mesh = plsc.VectorSubcoreMesh(core_axis_name="c", subcore_axis_name="s")
    fn = functools.partial(
        pl.kernel,
        mesh=mesh,
        out_type=jax.ShapeDtypeStruct((PMAX, D), jnp.float32),
        scratch_types=[
            pltpu.VMEM((TW,), jnp.int32),
            pltpu.VMEM((TW,), jnp.int32),
            pltpu.VMEM((TW, D), jnp.float32),
            pltpu.SemaphoreType.DMA,
        ],
    )(_sc_scatter_body)
    return fn(xf, r0, r1)


def _ffn_body(gid_ref, nu_ref, x_ref, w1_ref, b1_ref, w2_ref, b2_ref, y_ref):
    i = pl.program_id(0)

    @pl.when(i < nu_ref[0])
    def _():
        h = jnp.dot(x_ref[...], w1_ref[0], preferred_element_type=jnp.float32) + b1_ref[0]
        h = 0.5 * h * (1.0 + lax.erf(h * SQRT1_2))
        y_ref[...] = jnp.dot(h, w2_ref[0], preferred_element_type=jnp.float32) + b2_ref[0]


def _ffn_call(gid, nu, xg, W1, b1, W2, b2, interpret=False):
    grid_spec = pltpu.PrefetchScalarGridSpec(
        num_scalar_prefetch=2,
        grid=(NB,),
        in_specs=[
            pl.BlockSpec((BM, D), lambda i, gid, nu: (i, 0)),
            pl.BlockSpec((1, D, F), lambda i, gid, nu: (gid[i], 0, 0)),
            pl.BlockSpec((1, 1, F), lambda i, gid, nu: (gid[i], 0, 0)),
            pl.BlockSpec((1, F, D), lambda i, gid, nu: (gid[i], 0, 0)),
            pl.BlockSpec((1, 1, D), lambda i, gid, nu: (gid[i], 0, 0)),
        ],
        out_specs=pl.BlockSpec((BM, D), lambda i, gid, nu: (i, 0)),
    )
    return pl.pallas_call(
        _ffn_body,
        grid_spec=grid_spec,
        out_shape=jax.ShapeDtypeStruct((PMAX, D), jnp.float32),
        interpret=interpret,
    )(gid, nu, xg, W1, b1.reshape(E, 1, F), W2, b2.reshape(E, 1, D))


def _sc_combine_body(y_hbm, r0_hbm, r1_hbm, w0_hbm, w1_hbm, out_hbm,
                     i0v, i1v, w0v, w1v, y0v, y1v, sem):
    wid = lax.axis_index("s") * NC + lax.axis_index("c")
    base = wid * TW
    for c in range(TW // CW):
        cb = base + c * CW
        pltpu.sync_copy(r0_hbm.at[pl.ds(cb, CW)], i0v)
        pltpu.sync_copy(r1_hbm.at[pl.ds(cb, CW)], i1v)
        pltpu.sync_copy(w0_hbm.at[pl.ds(cb, CW)], w0v)
        pltpu.sync_copy(w1_hbm.at[pl.ds(cb, CW)], w1v)
        pltpu.async_copy(y_hbm.at[i0v], y0v, sem).wait()
        pltpu.async_copy(y_hbm.at[i1v], y1v, sem).wait()

        def tok_body(i, _):
            a = w0v[i, :]
            b = w1v[i, :]

            def col_body(j, _):
                sl = pl.ds(j * 16, 16)
                y0v[i, sl] = y0v[i, sl] * a + y1v[i, sl] * b
                return 0

            lax.fori_loop(0, D // 16, col_body, 0)
            return 0

        lax.fori_loop(0, CW, tok_body, 0)
        pltpu.sync_copy(y0v, out_hbm.at[pl.ds(cb, CW)])


def _sc_combine(y, r0, r1, w0r, w1r):
    mesh = plsc.VectorSubcoreMesh(core_axis_name="c", subcore_axis_name="s")
    fn = functools.partial(
        pl.kernel,
        mesh=mesh,
        out_type=jax.ShapeDtypeStruct((T, D), jnp.float32),
        scratch_types=[
            pltpu.VMEM((CW,), jnp.int32),
            pltpu.VMEM((CW,), jnp.int32),
            pltpu.VMEM((CW, 16), jnp.float32),
            pltpu.VMEM((CW, 16), jnp.float32),
            pltpu.VMEM((CW, D), jnp.float32),
            pltpu.VMEM((CW, D), jnp.float32),
            pltpu.SemaphoreType.DMA,
        ],
    )(_sc_combine_body)
    return fn(y, r0, r1, w0r, w1r)


def kernel(x, Wr, W1, b1, W2, b2):
    B, S, d = x.shape
    xf = x.reshape(B * S, d)
    Wrp = jnp.pad(Wr, ((0, 0), (0, EP - E)))
    r0, r1, w0r, w1r, gid, nu = _router_call(xf, Wrp)
    r0 = r0.reshape(T)
    r1 = r1.reshape(T)
    xg = _sc_scatter(xf, r0, r1)
    y = _ffn_call(gid[:NB, 0], nu.reshape(1), xg, W1, b1, W2, b2)
    out = _sc_combine(y, r0, r1, w0r, w1r)
    return out.reshape(B, S, d), jnp.array(0.0, dtype=x.dtype)


# speedup vs baseline: 1.2492x; 1.0320x over previous
import functools

import jax
import jax.numpy as jnp
from jax import lax
from jax.experimental import pallas as pl
from jax.experimental.pallas import tpu as pltpu
from jax.experimental.pallas import tpu_sc as plsc

D = 1024
E = 8
EP = 128
F = 2048
T = 2048
K = 2
BM = 256
NB = 23
PMAX = NB * BM
SQRT1_2 = 0.7071067811865476

NC, NS = 2, 16
NW = NC * NS
TW = T // NW
CW = 32


def _router_body(x_ref, wr_ref, r0_ref, r1_ref, w0_ref, w1_ref, gid_ref, nu_ref):
    xx = x_ref[...]
    L = jnp.dot(xx, wr_ref[...], preferred_element_type=jnp.float32)
    idx = lax.broadcasted_iota(jnp.int32, (T, EP), 1)
    L = jnp.where(idx < E, L, -1e30)
    m1 = jnp.max(L, axis=1, keepdims=True)
    e0 = jnp.min(jnp.where(L >= m1, idx, 2**30), axis=1, keepdims=True)
    L2 = jnp.where(idx == e0, -1e30, L)
    m2 = jnp.max(L2, axis=1, keepdims=True)
    e1 = jnp.min(jnp.where(L2 >= m2, idx, 2**30), axis=1, keepdims=True)
    w0 = 1.0 / (1.0 + jnp.exp(m2 - m1))
    w1 = 1.0 - w0

    oh0 = (idx == e0).astype(jnp.float32)
    oh1 = (idx == e1).astype(jnp.float32)
    mask = oh0 + oh1

    ri = lax.broadcasted_iota(jnp.int32, (BM, BM), 0)
    ci = lax.broadcasted_iota(jnp.int32, (BM, BM), 1)
    tril = (ri > ci).astype(jnp.float32)
    parts = []
    carry = jnp.zeros((1, EP), dtype=jnp.float32)
    for i in range(T // BM):
        blk = mask[i * BM:(i + 1) * BM, :]
        parts.append(jnp.dot(tril, blk, preferred_element_type=jnp.float32) + carry)
        carry = carry + jnp.sum(blk, axis=0, keepdims=True)
    rank = jnp.concatenate(parts, axis=0)
    n = carry

    nb = jnp.floor((n + (BM - 1)) / BM)
    padded = nb * BM
    li = lax.broadcasted_iota(jnp.int32, (EP, EP), 0)
    lj = lax.broadcasted_iota(jnp.int32, (EP, EP), 1)
    u_lt = (li < lj).astype(jnp.float32)
    u_le = (li <= lj).astype(jnp.float32)
    off = jnp.dot(padded, u_lt, preferred_element_type=jnp.float32)
    cnb = jnp.dot(nb, u_le, preferred_element_type=jnp.float32)

    dest = off + rank
    r0 = jnp.sum(oh0 * dest, axis=1, keepdims=True)
    r1 = jnp.sum(oh1 * dest, axis=1, keepdims=True)
    r0_ref[...] = r0.astype(jnp.int32)
    r1_ref[...] = r1.astype(jnp.int32)
    w0_ref[...] = jnp.broadcast_to(w0, (T, 16))
    w1_ref[...] = jnp.broadcast_to(w1, (T, 16))

    brow = lax.broadcasted_iota(jnp.int32, (EP, EP), 0).astype(jnp.float32)
    cmp = (jnp.broadcast_to(cnb, (EP, EP)) <= brow).astype(jnp.float32)
    g = jnp.sum(cmp, axis=1, keepdims=True)
    lane = lax.broadcasted_iota(jnp.int32, (1, EP), 1)
    g_last = jnp.max(jnp.where((n > 0) & (lane < E), lane, 0), axis=1, keepdims=True)
    g = jnp.minimum(g, g_last.astype(jnp.float32))
    gid_ref[...] = g.astype(jnp.int32)
    nu_ref[...] = jnp.sum(nb, axis=1, keepdims=True).astype(jnp.int32)


def _router_call(xf, Wrp, interpret=False):
    return pl.pallas_call(
        _router_body,
        out_shape=[
            jax.ShapeDtypeStruct((T, 1), jnp.int32),
            jax.ShapeDtypeStruct((T, 1), jnp.int32),
            jax.ShapeDtypeStruct((T, 16), jnp.float32),
            jax.ShapeDtypeStruct((T, 16), jnp.float32),
            jax.ShapeDtypeStruct((EP, 1), jnp.int32),
            jax.ShapeDtypeStruct((1, 1), jnp.int32),
        ],
        interpret=interpret,
    )(xf, Wrp)


def _sc_scatter_body(x_hbm, r0_hbm, r1_hbm, xg_hbm, i0v, i1v, xv, sem, sem2):
    wid = lax.axis_index("s") * NC + lax.axis_index("c")
    base = wid * TW
    pltpu.sync_copy(x_hbm.at[pl.ds(base, TW)], xv)
    pltpu.sync_copy(r0_hbm.at[pl.ds(base, TW)], i0v)
    pltpu.sync_copy(r1_hbm.at[pl.ds(base, TW)], i1v)
    c0 = pltpu.async_copy(xv, xg_hbm.at[i0v], sem)
    c1 = pltpu.async_copy(xv, xg_hbm.at[i1v], sem2)
    c0.wait()
    c1.wait()


def _sc_scatter(xf, r0, r1):
    mesh = plsc.VectorSubcoreMesh(core_axis_name="c", subcore_axis_name="s")
    fn = functools.partial(
        pl.kernel,
        mesh=mesh,
        out_type=jax.ShapeDtypeStruct((PMAX, D), jnp.float32),
        scratch_types=[
            pltpu.VMEM((TW,), jnp.int32),
            pltpu.VMEM((TW,), jnp.int32),
            pltpu.VMEM((TW, D), jnp.float32),
            pltpu.SemaphoreType.DMA,
            pltpu.SemaphoreType.DMA,
        ],
    )(_sc_scatter_body)
    return fn(xf, r0, r1)


def _ffn_body(gid_ref, nu_ref, x_ref, w1_ref, b1_ref, w2_ref, b2_ref, y_ref):
    i = pl.program_id(0)

    @pl.when(i < nu_ref[0])
    def _():
        h = jnp.dot(x_ref[...], w1_ref[0], preferred_element_type=jnp.float32) + b1_ref[0]
        h = 0.5 * h * (1.0 + lax.erf(h * SQRT1_2))
        y_ref[...] = jnp.dot(h, w2_ref[0], preferred_element_type=jnp.float32) + b2_ref[0]


def _ffn_call(gid, nu, xg, W1, b1, W2, b2, interpret=False):
    grid_spec = pltpu.PrefetchScalarGridSpec(
        num_scalar_prefetch=2,
        grid=(NB,),
        in_specs=[
            pl.BlockSpec((BM, D), lambda i, gid, nu: (i, 0)),
            pl.BlockSpec((1, D, F), lambda i, gid, nu: (gid[i], 0, 0)),
            pl.BlockSpec((1, 1, F), lambda i, gid, nu: (gid[i], 0, 0)),
            pl.BlockSpec((1, F, D), lambda i, gid, nu: (gid[i], 0, 0)),
            pl.BlockSpec((1, 1, D), lambda i, gid, nu: (gid[i], 0, 0)),
        ],
        out_specs=pl.BlockSpec((BM, D), lambda i, gid, nu: (i, 0)),
    )
    return pl.pallas_call(
        _ffn_body,
        grid_spec=grid_spec,
        out_shape=jax.ShapeDtypeStruct((PMAX, D), jnp.float32),
        interpret=interpret,
    )(gid, nu, xg, W1, b1.reshape(E, 1, F), W2, b2.reshape(E, 1, D))


def _sc_combine_body(y_hbm, r0_hbm, r1_hbm, w0_hbm, w1_hbm, out_hbm,
                     i0v, i1v, w0v, w1v, y0v, y1v, sem, sem2):
    wid = lax.axis_index("s") * NC + lax.axis_index("c")
    base = wid * TW
    for c in range(TW // CW):
        cb = base + c * CW
        pltpu.sync_copy(r0_hbm.at[pl.ds(cb, CW)], i0v)
        pltpu.sync_copy(r1_hbm.at[pl.ds(cb, CW)], i1v)
        pltpu.sync_copy(w0_hbm.at[pl.ds(cb, CW)], w0v)
        pltpu.sync_copy(w1_hbm.at[pl.ds(cb, CW)], w1v)
        c0 = pltpu.async_copy(y_hbm.at[i0v], y0v, sem)
        c1 = pltpu.async_copy(y_hbm.at[i1v], y1v, sem2)
        c0.wait()
        c1.wait()

        def tok_body(i, _):
            a = w0v[i, :]
            b = w1v[i, :]

            def col_body(j, _):
                sl = pl.ds(j * 16, 16)
                y0v[i, sl] = y0v[i, sl] * a + y1v[i, sl] * b
                return 0

            lax.fori_loop(0, D // 16, col_body, 0, unroll=8)
            return 0

        lax.fori_loop(0, CW, tok_body, 0)
        pltpu.sync_copy(y0v, out_hbm.at[pl.ds(cb, CW)])


def _sc_combine(y, r0, r1, w0r, w1r):
    mesh = plsc.VectorSubcoreMesh(core_axis_name="c", subcore_axis_name="s")
    fn = functools.partial(
        pl.kernel,
        mesh=mesh,
        out_type=jax.ShapeDtypeStruct((T, D), jnp.float32),
        scratch_types=[
            pltpu.VMEM((CW,), jnp.int32),
            pltpu.VMEM((CW,), jnp.int32),
            pltpu.VMEM((CW, 16), jnp.float32),
            pltpu.VMEM((CW, 16), jnp.float32),
            pltpu.VMEM((CW, D), jnp.float32),
            pltpu.VMEM((CW, D), jnp.float32),
            pltpu.SemaphoreType.DMA,
            pltpu.SemaphoreType.DMA,
        ],
    )(_sc_combine_body)
    return fn(y, r0, r1, w0r, w1r)


def kernel(x, Wr, W1, b1, W2, b2):
    B, S, d = x.shape
    xf = x.reshape(B * S, d)
    Wrp = jnp.pad(Wr, ((0, 0), (0, EP - E)))
    r0, r1, w0r, w1r, gid, nu = _router_call(xf, Wrp)
    r0 = r0.reshape(T)
    r1 = r1.reshape(T)
    xg = _sc_scatter(xf, r0, r1)
    y = _ffn_call(gid[:NB, 0], nu.reshape(1), xg, W1, b1, W2, b2)
    out = _sc_combine(y, r0, r1, w0r, w1r)
    return out.reshape(B, S, d), jnp.array(0.0, dtype=x.dtype)


# speedup vs baseline: 1.3839x; 1.1078x over previous
import functools

import jax
import jax.numpy as jnp
from jax import lax
from jax.experimental import pallas as pl
from jax.experimental.pallas import tpu as pltpu
from jax.experimental.pallas import tpu_sc as plsc

D = 1024
E = 8
EP = 128
F = 2048
T = 2048
K = 2
BM = 256
NB = 23
PMAX = NB * BM
SQRT1_2 = 0.7071067811865476

NC, NS = 2, 16
NW = NC * NS
TW = T // NW
CW = 32


def _router_body(x_ref, wr_ref, r0_ref, r1_ref, w0_ref, w1_ref, gid_ref, nu_ref):
    xx = x_ref[...]
    L = jnp.dot(xx, wr_ref[...], preferred_element_type=jnp.float32)
    idx = lax.broadcasted_iota(jnp.int32, (T, EP), 1)
    L = jnp.where(idx < E, L, -1e30)
    m1 = jnp.max(L, axis=1, keepdims=True)
    e0 = jnp.min(jnp.where(L >= m1, idx, 2**30), axis=1, keepdims=True)
    L2 = jnp.where(idx == e0, -1e30, L)
    m2 = jnp.max(L2, axis=1, keepdims=True)
    e1 = jnp.min(jnp.where(L2 >= m2, idx, 2**30), axis=1, keepdims=True)
    w0 = 1.0 / (1.0 + jnp.exp(m2 - m1))
    w1 = 1.0 - w0

    oh0 = (idx == e0).astype(jnp.float32)
    oh1 = (idx == e1).astype(jnp.float32)
    mask = oh0 + oh1

    ri = lax.broadcasted_iota(jnp.int32, (BM, BM), 0)
    ci = lax.broadcasted_iota(jnp.int32, (BM, BM), 1)
    tril = (ri > ci).astype(jnp.float32)
    parts = []
    carry = jnp.zeros((1, EP), dtype=jnp.float32)
    for i in range(T // BM):
        blk = mask[i * BM:(i + 1) * BM, :]
        parts.append(jnp.dot(tril, blk, preferred_element_type=jnp.float32) + carry)
        carry = carry + jnp.sum(blk, axis=0, keepdims=True)
    rank = jnp.concatenate(parts, axis=0)
    n = carry

    nb = jnp.floor((n + (BM - 1)) / BM)
    padded = nb * BM
    li = lax.broadcasted_iota(jnp.int32, (EP, EP), 0)
    lj = lax.broadcasted_iota(jnp.int32, (EP, EP), 1)
    u_lt = (li < lj).astype(jnp.float32)
    u_le = (li <= lj).astype(jnp.float32)
    off = jnp.dot(padded, u_lt, preferred_element_type=jnp.float32)
    cnb = jnp.dot(nb, u_le, preferred_element_type=jnp.float32)

    dest = off + rank
    r0 = jnp.sum(oh0 * dest, axis=1, keepdims=True)
    r1 = jnp.sum(oh1 * dest, axis=1, keepdims=True)
    r0_ref[...] = r0.astype(jnp.int32)
    r1_ref[...] = r1.astype(jnp.int32)
    w0_ref[...] = jnp.broadcast_to(w0, (T, 128))
    w1_ref[...] = jnp.broadcast_to(w1, (T, 128))

    brow = lax.broadcasted_iota(jnp.int32, (EP, EP), 0).astype(jnp.float32)
    cmp = (jnp.broadcast_to(cnb, (EP, EP)) <= brow).astype(jnp.float32)
    g = jnp.sum(cmp, axis=1, keepdims=True)
    lane = lax.broadcasted_iota(jnp.int32, (1, EP), 1)
    g_last = jnp.max(jnp.where((n > 0) & (lane < E), lane, 0), axis=1, keepdims=True)
    g = jnp.minimum(g, g_last.astype(jnp.float32))
    gid_ref[...] = g.astype(jnp.int32)
    nu_ref[...] = jnp.sum(nb, axis=1, keepdims=True).astype(jnp.int32)


def _router_call(xf, Wrp, interpret=False):
    return pl.pallas_call(
        _router_body,
        out_shape=[
            jax.ShapeDtypeStruct((T, 1), jnp.int32),
            jax.ShapeDtypeStruct((T, 1), jnp.int32),
            jax.ShapeDtypeStruct((T, 128), jnp.float32),
            jax.ShapeDtypeStruct((T, 128), jnp.float32),
            jax.ShapeDtypeStruct((EP, 1), jnp.int32),
            jax.ShapeDtypeStruct((1, 1), jnp.int32),
        ],
        interpret=interpret,
    )(xf, Wrp)


def _sc_scatter_body(x_hbm, r0_hbm, r1_hbm, w0_hbm, w1_hbm, xg_hbm, wrow_hbm,
                     i0v, i1v, xv, w0v, w1v, sem, sem2, sem3, sem4):
    wid = lax.axis_index("s") * NC + lax.axis_index("c")
    base = wid * TW
    pltpu.sync_copy(x_hbm.at[pl.ds(base, TW)], xv)
    pltpu.sync_copy(r0_hbm.at[pl.ds(base, TW)], i0v)
    pltpu.sync_copy(r1_hbm.at[pl.ds(base, TW)], i1v)
    pltpu.sync_copy(w0_hbm.at[pl.ds(base, TW)], w0v)
    pltpu.sync_copy(w1_hbm.at[pl.ds(base, TW)], w1v)
    c0 = pltpu.async_copy(xv, xg_hbm.at[i0v], sem)
    c1 = pltpu.async_copy(xv, xg_hbm.at[i1v], sem2)
    c2 = pltpu.async_copy(w0v, wrow_hbm.at[i0v], sem3)
    c3 = pltpu.async_copy(w1v, wrow_hbm.at[i1v], sem4)
    c0.wait()
    c1.wait()
    c2.wait()
    c3.wait()


def _sc_scatter(xf, r0, r1, w0r, w1r):
    mesh = plsc.VectorSubcoreMesh(core_axis_name="c", subcore_axis_name="s")
    fn = functools.partial(
        pl.kernel,
        mesh=mesh,
        out_type=[
            jax.ShapeDtypeStruct((PMAX, D), jnp.float32),
            jax.ShapeDtypeStruct((PMAX, 128), jnp.float32),
        ],
        scratch_types=[
            pltpu.VMEM((TW,), jnp.int32),
            pltpu.VMEM((TW,), jnp.int32),
            pltpu.VMEM((TW, D), jnp.float32),
            pltpu.VMEM((TW, 128), jnp.float32),
            pltpu.VMEM((TW, 128), jnp.float32),
            pltpu.SemaphoreType.DMA,
            pltpu.SemaphoreType.DMA,
            pltpu.SemaphoreType.DMA,
            pltpu.SemaphoreType.DMA,
        ],
    )(_sc_scatter_body)
    return fn(xf, r0, r1, w0r, w1r)


def _ffn_body(gid_ref, nu_ref, x_ref, w1_ref, b1_ref, w2_ref, b2_ref, wr_ref, y_ref):
    i = pl.program_id(0)

    @pl.when(i < nu_ref[0])
    def _():
        h = jnp.dot(x_ref[...], w1_ref[0], preferred_element_type=jnp.float32) + b1_ref[0]
        h = 0.5 * h * (1.0 + lax.erf(h * SQRT1_2))
        y = jnp.dot(h, w2_ref[0], preferred_element_type=jnp.float32) + b2_ref[0]
        y_ref[...] = y * wr_ref[:, :1]


def _ffn_call(gid, nu, xg, wrow, W1, b1, W2, b2, interpret=False):
    grid_spec = pltpu.PrefetchScalarGridSpec(
        num_scalar_prefetch=2,
        grid=(NB,),
        in_specs=[
            pl.BlockSpec((BM, D), lambda i, gid, nu: (i, 0)),
            pl.BlockSpec((1, D, F), lambda i, gid, nu: (gid[i], 0, 0)),
            pl.BlockSpec((1, 1, F), lambda i, gid, nu: (gid[i], 0, 0)),
            pl.BlockSpec((1, F, D), lambda i, gid, nu: (gid[i], 0, 0)),
            pl.BlockSpec((1, 1, D), lambda i, gid, nu: (gid[i], 0, 0)),
            pl.BlockSpec((BM, 128), lambda i, gid, nu: (i, 0)),
        ],
        out_specs=pl.BlockSpec((BM, D), lambda i, gid, nu: (i, 0)),
    )
    return pl.pallas_call(
        _ffn_body,
        grid_spec=grid_spec,
        out_shape=jax.ShapeDtypeStruct((PMAX, D), jnp.float32),
        interpret=interpret,
    )(gid, nu, xg, W1, b1.reshape(E, 1, F), W2, b2.reshape(E, 1, D), wrow)


def _sc_combine_body(y_hbm, r0_hbm, r1_hbm, out_hbm,
                     i0v, i1v, y0v, y1v, sem, sem2):
    wid = lax.axis_index("s") * NC + lax.axis_index("c")
    base = wid * TW
    for c in range(TW // CW):
        cb = base + c * CW
        pltpu.sync_copy(r0_hbm.at[pl.ds(cb, CW)], i0v)
        pltpu.sync_copy(r1_hbm.at[pl.ds(cb, CW)], i1v)
        c0 = pltpu.async_copy(y_hbm.at[i0v], y0v, sem)
        c1 = pltpu.async_copy(y_hbm.at[i1v], y1v, sem2)
        c0.wait()
        c1.wait()

        def tok_body(i, _):
            for j in range(D // 16):
                sl = pl.ds(j * 16, 16)
                y0v[i, sl] = y0v[i, sl] + y1v[i, sl]
            return 0

        lax.fori_loop(0, CW, tok_body, 0)
        pltpu.sync_copy(y0v, out_hbm.at[pl.ds(cb, CW)])


def _sc_combine(y, r0, r1):
    mesh = plsc.VectorSubcoreMesh(core_axis_name="c", subcore_axis_name="s")
    fn = functools.partial(
        pl.kernel,
        mesh=mesh,
        out_type=jax.ShapeDtypeStruct((T, D), jnp.float32),
        scratch_types=[
            pltpu.VMEM((CW,), jnp.int32),
            pltpu.VMEM((CW,), jnp.int32),
            pltpu.VMEM((CW, D), jnp.float32),
            pltpu.VMEM((CW, D), jnp.float32),
            pltpu.SemaphoreType.DMA,
            pltpu.SemaphoreType.DMA,
        ],
    )(_sc_combine_body)
    return fn(y, r0, r1)


def kernel(x, Wr, W1, b1, W2, b2):
    B, S, d = x.shape
    xf = x.reshape(B * S, d)
    Wrp = jnp.pad(Wr, ((0, 0), (0, EP - E)))
    r0, r1, w0r, w1r, gid, nu = _router_call(xf, Wrp)
    r0 = r0.reshape(T)
    r1 = r1.reshape(T)
    xg, wrow = _sc_scatter(xf, r0, r1, w0r, w1r)
    y = _ffn_call(gid[:NB, 0], nu.reshape(1), xg, wrow, W1, b1, W2, b2)
    out = _sc_combine(y, r0, r1)
    return out.reshape(B, S, d), jnp.array(0.0, dtype=x.dtype)
